# baseline (speedup 1.0000x reference)
import functools

import jax
import jax.numpy as jnp
from jax import lax
from jax.experimental import pallas as pl
from jax.experimental.pallas import tpu as pltpu
from jax.experimental.pallas import tpu_sc as plsc

B = 4096
L = 200
D = 32
N_CATE = 1000

NC = 2
NS = 16
NW = NC * NS
BPW = B // NW
HALF = BPW // 2
NFULL = 12
EPI_OFF = L - 16
HPAD = 1008
TC_BLK = 512

_mesh = plsc.VectorSubcoreMesh(core_axis_name="c", subcore_axis_name="s")

_GATHER_DNUMS = lax.GatherDimensionNumbers(
    offset_dims=(), collapsed_slice_dims=(0,), start_index_map=(0,))


def _bcast_lane(v, sel):
    return lax.gather(v, sel[:, None], _GATHER_DNUMS, (1,),
                      mode=lax.GatherScatterMode.PROMISE_IN_BOUNDS)


@functools.partial(
    pl.kernel,
    out_type=[jax.ShapeDtypeStruct((B, D), jnp.float32) for _ in range(4)]
    + [jax.ShapeDtypeStruct((B, N_CATE), jnp.float32)],
    mesh=_mesh,
    compiler_params=pltpu.CompilerParams(
        needs_layout_passes=False, use_tc_tiling_on_sc=False),
    scratch_types=[
        pltpu.VMEM((HALF, L), jnp.int32),
        pltpu.VMEM((HALF, L), jnp.int32),
        pltpu.VMEM((HALF, L), jnp.float32),
        pltpu.VMEM((L, D), jnp.float32),
        pltpu.VMEM((L, D), jnp.float32),
        pltpu.VMEM((HPAD,), jnp.float32),
        pltpu.VMEM((HPAD,), jnp.float32),
        pltpu.VMEM((BPW,), jnp.int32),
        pltpu.VMEM((BPW, D), jnp.float32),
        pltpu.VMEM((HALF, D), jnp.float32),
        pltpu.SemaphoreType.DMA,
        pltpu.SemaphoreType.DMA,
        pltpu.SemaphoreType.DMA,
        pltpu.SemaphoreType.DMA,
        pltpu.SemaphoreType.DMA,
    ],
)
def _din_sc_kernel(
    uid_idx, mid_idx, cate_idx, mid_his, cate_his, mask,
    uid_tab, mid_tab, cate_tab,
    out_uid, out_mid, out_cate, out_msum, out_hist,
    midx_v, cidx_v, mask_v, rows0_v, rows1_v, hist0_v, hist1_v,
    sidx_v, srow_v, msum_v, sem, sem0, sem1, hsem0, hsem1,
):
    wid = lax.axis_index("s") * NC + lax.axis_index("c")
    base = wid * BPW
    iota16 = lax.iota(jnp.int32, 16)

    for idx_hbm, tab_hbm, out_hbm in (
        (uid_idx, uid_tab, out_uid),
        (mid_idx, mid_tab, out_mid),
        (cate_idx, cate_tab, out_cate),
    ):
        pltpu.sync_copy(idx_hbm.at[pl.ds(base, BPW)], sidx_v)
        pltpu.async_copy(tab_hbm.at[sidx_v], srow_v, sem).wait()
        pltpu.sync_copy(srow_v, out_hbm.at[pl.ds(base, BPW)])

    def row_copies(r, rows_ref, sem_ref):
        return (
            (mid_tab.at[midx_v.at[r, pl.ds(0, 128)]],
             rows_ref.at[pl.ds(0, 128)], sem_ref),
            (mid_tab.at[midx_v.at[r, pl.ds(128, L - 128)]],
             rows_ref.at[pl.ds(128, L - 128)], sem_ref),
        )

    def fire_row(r, rows_ref, sem_ref):
        for src, dst, s in row_copies(r, rows_ref, sem_ref):
            pltpu.async_copy(src, dst, s)

    def wait_row(r, rows_ref, sem_ref):
        for src, dst, s in row_copies(r, rows_ref, sem_ref):
            pltpu.make_async_copy(src, dst, s).wait()

    def lanes_accum(rows_ref, mv, off, js, accs):
        accs = list(accs)
        for j in js:
            sel = jnp.full((16,), j, dtype=jnp.int32)
            bm = _bcast_lane(mv, sel)
            lrow = off + j
            m0 = rows_ref[lrow, pl.ds(0, 16)]
            m1 = rows_ref[lrow, pl.ds(16, 16)]
            p = 2 * (j % 2)
            accs[p + 0] = accs[p + 0] + bm * m0
            accs[p + 1] = accs[p + 1] + bm * m1
        return tuple(accs)

    zero = jnp.zeros((16,), jnp.float32)

    def compute_row(r, rows_ref, hist_ref):
        zeros4 = (zero,) * 4

        for k in range(HPAD // 16):
            hist_ref[pl.ds(k * 16, 16)] = zero

        def chunk_body(c, carry):
            off = c * 16
            mv = mask_v[r, pl.ds(off, 16)]
            civ = cidx_v[r, pl.ds(off, 16)]
            plsc.addupdate_scatter(hist_ref, [civ], mv)
            return lanes_accum(rows_ref, mv, off, range(16), carry)

        accs = lax.fori_loop(0, NFULL, chunk_body, zeros4)

        mv = mask_v[r, pl.ds(EPI_OFF, 16)]
        civ = cidx_v[r, pl.ds(EPI_OFF, 16)]
        plsc.addupdate_scatter(hist_ref, [civ], mv, mask=iota16 >= 8)
        accs = lanes_accum(rows_ref, mv, EPI_OFF, range(8, 16), accs)

        msum_v[r, pl.ds(0, 16)] = accs[0] + accs[2]
        msum_v[r, pl.ds(16, 16)] = accs[1] + accs[3]

    def hist_copy(glob_r, hist_ref, hsem):
        return pltpu.make_async_copy(
            hist_ref.at[pl.ds(0, N_CATE)], out_hist.at[glob_r], hsem)

    def pair_body(row0, i, _):
        r0 = 2 * i
        r1 = r0 + 1
        fire_row(r1, rows1_v, sem1)
        wait_row(r0, rows0_v, sem0)

        @pl.when(i > 0)
        def _():
            hist_copy(row0 + r0 - 2, hist0_v, hsem0).wait()

        compute_row(r0, rows0_v, hist0_v)
        pltpu.async_copy(hist0_v.at[pl.ds(0, N_CATE)],
                         out_hist.at[row0 + r0], hsem0)

        @pl.when(r0 + 2 < HALF)
        def _():
            fire_row(r0 + 2, rows0_v, sem0)

        wait_row(r1, rows1_v, sem1)

        @pl.when(i > 0)
        def _():
            hist_copy(row0 + r1 - 2, hist1_v, hsem1).wait()

        compute_row(r1, rows1_v, hist1_v)
        pltpu.async_copy(hist1_v.at[pl.ds(0, N_CATE)],
                         out_hist.at[row0 + r1], hsem1)
        return 0

    for h in range(2):
        row0 = base + h * HALF
        pltpu.sync_copy(mid_his.at[pl.ds(row0, HALF), :], midx_v)
        pltpu.sync_copy(cate_his.at[pl.ds(row0, HALF), :], cidx_v)
        pltpu.sync_copy(mask.at[pl.ds(row0, HALF), :], mask_v)
        fire_row(0, rows0_v, sem0)
        lax.fori_loop(0, HALF // 2, functools.partial(pair_body, row0), 0)
        hist_copy(row0 + HALF - 2, hist0_v, hsem0).wait()
        hist_copy(row0 + HALF - 1, hist1_v, hsem1).wait()
        pltpu.sync_copy(msum_v, out_msum.at[pl.ds(row0, HALF)])


def _csum_tc_body(hist_ref, tab_ref, out_ref):
    out_ref[...] = jnp.dot(hist_ref[...], tab_ref[...],
                           preferred_element_type=jnp.float32)


def _csum_tc(hist, cate_table):
    return pl.pallas_call(
        _csum_tc_body,
        grid=(B // TC_BLK,),
        in_specs=[
            pl.BlockSpec((TC_BLK, N_CATE), lambda i: (i, 0)),
            pl.BlockSpec((N_CATE, D), lambda i: (0, 0)),
        ],
        out_specs=pl.BlockSpec((TC_BLK, D), lambda i: (i, 0)),
        out_shape=jax.ShapeDtypeStruct((B, D), jnp.float32),
    )(hist, cate_table)


def kernel(uid_batch, mid_batch, cate_batch, mid_his_batch, cate_his_batch,
           mask, uid_table, mid_table, cate_table):
    o_uid, o_mid, o_cate, o_msum, o_hist = _din_sc_kernel(
        uid_batch.astype(jnp.int32), mid_batch.astype(jnp.int32),
        cate_batch.astype(jnp.int32), mid_his_batch.astype(jnp.int32),
        cate_his_batch.astype(jnp.int32), mask,
        uid_table, mid_table, cate_table)
    o_csum = _csum_tc(o_hist, cate_table)
    return jnp.concatenate([o_uid, o_mid, o_cate, o_msum, o_csum], axis=1)
